# baseline (speedup 1.0000x reference)
import functools

import jax
import jax.numpy as jnp
from jax import lax
from jax.experimental import pallas as pl
from jax.experimental.pallas import tpu as pltpu
from jax.experimental.pallas import tpu_sc as plsc

_F = 32
_TL = 128
_LANES = 16


def _make_fpmc(L):
    P = 80
    C = 10

    @functools.partial(
        pl.kernel,
        out_type=jax.ShapeDtypeStruct((_LANES,), jnp.float32),
        scratch_types=[
            pltpu.VMEM((P,), jnp.int32),
            pltpu.VMEM((C, _F, _TL), jnp.float32),
            pltpu.VMEM((C, _F, _TL), jnp.float32),
            pltpu.VMEM((_F, _TL), jnp.float32),
            pltpu.VMEM((_F, _TL), jnp.float32),
            pltpu.VMEM((_F, _TL), jnp.float32),
            pltpu.VMEM((_F, _TL), jnp.float32),
            pltpu.VMEM((_LANES,), jnp.float32),
            pltpu.SemaphoreType.DMA,
            pltpu.SemaphoreType.DMA,
        ],
        mesh=plsc.VectorSubcoreMesh(core_axis_name="c", subcore_axis_name="s"),
        compiler_params=pltpu.CompilerParams(needs_layout_passes=False),
    )
    def fpmc(packed_hbm,
             v_il, v_li, v_ul, v_lu, v_ui, v_iu,
             out_hbm,
             idx_v,
             blks_li, blks_lu, blk_il, blk_iu, blk_ul, blk_ui,
             res_v, sem, sem2):
        cid = lax.axis_index("c")
        sid = lax.axis_index("s")

        @pl.when(jnp.logical_and(cid == 0, sid == 0))
        def _():
            pltpu.sync_copy(packed_hbm, idx_v)
            vs = [idx_v[pl.ds(16 * b, 16)] for b in range(P // 16)]

            def sidx(pos):
                return vs[pos // 16][pos % 16] - 1

            def tile_copy(src, dst, idx, s=None):
                base = pl.multiple_of((idx // _TL) * _TL, _TL)
                return pltpu.async_copy(
                    src.at[:, pl.ds(base, _TL)], dst, s if s is not None else sem)

            i0 = sidx(56)
            u0 = sidx(64)
            singles = [
                tile_copy(v_il, blk_il, i0, sem2),
                tile_copy(v_iu, blk_iu, i0, sem2),
                tile_copy(v_ul, blk_ul, u0, sem2),
                tile_copy(v_ui, blk_ui, u0, sem2),
            ]

            rows16 = lax.iota(jnp.int32, _LANES)

            def col_halves(blk, idx):
                lane = jnp.full((_LANES,), idx % _TL, jnp.int32)
                a = plsc.load_gather(blk, [rows16, lane])
                b = plsc.load_gather(blk, [rows16 + 16, lane])
                return a, b

            acc_li_a = jnp.zeros((_LANES,), jnp.float32)
            acc_li_b = jnp.zeros((_LANES,), jnp.float32)
            acc_lu_a = jnp.zeros((_LANES,), jnp.float32)
            acc_lu_b = jnp.zeros((_LANES,), jnp.float32)

            for c0 in range(0, L, C):
                n = min(C, L - c0)
                chunk = [
                    tile_copy(v_li, blks_li.at[j], sidx(c0 + j))
                    for j in range(n)
                ] + [
                    tile_copy(v_lu, blks_lu.at[j], sidx(c0 + j))
                    for j in range(n)
                ]
                for cp in chunk:
                    cp.wait()
                for j in range(n):
                    idx = sidx(c0 + j)
                    a, b = col_halves(blks_li.at[j], idx)
                    acc_li_a = acc_li_a + a
                    acc_li_b = acc_li_b + b
                    a, b = col_halves(blks_lu.at[j], idx)
                    acc_lu_a = acc_lu_a + a
                    acc_lu_b = acc_lu_b + b

            for cp in singles:
                cp.wait()
            vi_a, vi_b = col_halves(blk_il, i0)
            vu_a, vu_b = col_halves(blk_ul, u0)
            vui_a, vui_b = col_halves(blk_ui, u0)
            viu_a, viu_b = col_halves(blk_iu, i0)

            fac_s = jnp.where(vs[4][8] > 0,
                              jnp.float32(1.0 / L), jnp.float32(0.0))
            fac = jnp.full((_LANES,), fac_s, jnp.float32)
            r = (vi_a * acc_li_a + vi_b * acc_li_b
                 + vu_a * acc_lu_a + vu_b * acc_lu_b) * fac
            r = r + vui_a * viu_a + vui_b * viu_b
            lanes = lax.iota(jnp.int32, _LANES)
            res_v[...] = r
            for sh in (8, 4, 2, 1):
                r = r + plsc.load_gather(res_v, [lanes ^ sh])
                res_v[...] = r
            pltpu.sync_copy(res_v, out_hbm)

    return fpmc


def kernel(u, i, t, last_basket, V_IL, V_LI, V_UL, V_LU, V_UI, V_IU):
    L = last_basket.shape[0]
    lb = last_basket.astype(jnp.int32)
    packed = jnp.concatenate([
        lb,
        jnp.ones((56 - L,), jnp.int32),
        jnp.asarray(i, jnp.int32)[None],
        jnp.ones((7,), jnp.int32),
        jnp.asarray(u, jnp.int32)[None],
        jnp.ones((7,), jnp.int32),
        jnp.asarray(t, jnp.int32)[None],
        jnp.ones((7,), jnp.int32),
    ])
    out = _make_fpmc(L)(packed, V_IL.T, V_LI.T, V_UL.T, V_LU.T,
                        V_UI.T, V_IU.T)
    return out[0]


# speedup vs baseline: 8.8056x; 8.8056x over previous
import jax
import jax.numpy as jnp
from jax import lax
from jax.experimental import pallas as pl
from jax.experimental.pallas import tpu as pltpu

_F = 32
_TL = 128


def _fpmc_tc(L):
    def body(lb_ref, iarr, uarr, tarr,
             v_il, v_li, v_ul, v_lu, v_ui, v_iu, out_ref,
             blks_li, blks_lu, blk_il, blk_iu, blk_ul, blk_ui, sem):
        def tile_copy(src, dst, idx):
            base = pl.multiple_of((idx // _TL) * _TL, _TL)
            return pltpu.make_async_copy(
                src.at[:, pl.ds(base, _TL)], dst, sem)

        copies = []
        for l in range(L):
            idx = lb_ref[l] - 1
            copies.append(tile_copy(v_li, blks_li.at[l], idx))
            copies.append(tile_copy(v_lu, blks_lu.at[l], idx))
        i0 = iarr[0] - 1
        u0 = uarr[0] - 1
        copies.append(tile_copy(v_il, blk_il, i0))
        copies.append(tile_copy(v_iu, blk_iu, i0))
        copies.append(tile_copy(v_ul, blk_ul, u0))
        copies.append(tile_copy(v_ui, blk_ui, u0))
        for c in copies:
            c.start()
        for c in copies:
            c.wait()

        lane = lax.broadcasted_iota(jnp.int32, (_F, _TL), 1)

        def masked(blk, idx):
            return jnp.where(lane == idx % _TL, blk, 0.0)

        acc_li = masked(blks_li[0], lb_ref[0] - 1)
        acc_lu = masked(blks_lu[0], lb_ref[0] - 1)
        for l in range(1, L):
            idx = lb_ref[l] - 1
            acc_li = acc_li + masked(blks_li[l], idx)
            acc_lu = acc_lu + masked(blks_lu[l], idx)
        sum_li = jnp.sum(acc_li, axis=1)
        sum_lu = jnp.sum(acc_lu, axis=1)

        def col(blk, idx):
            return jnp.sum(masked(blk, idx), axis=1)

        vi = col(blk_il[...], i0)
        vu = col(blk_ul[...], u0)
        vui = col(blk_ui[...], u0)
        viu = col(blk_iu[...], i0)

        fac = jnp.where(tarr[0] > 0, jnp.float32(1.0 / L), jnp.float32(0.0))
        mc = (jnp.sum(sum_li * vi) + jnp.sum(sum_lu * vu)) * fac
        mf = jnp.sum(vui * viu)
        out_ref[0] = mc + mf

    grid_spec = pltpu.PrefetchScalarGridSpec(
        num_scalar_prefetch=4,
        grid=(),
        in_specs=[pl.BlockSpec(memory_space=pltpu.HBM)] * 6,
        out_specs=pl.BlockSpec(memory_space=pltpu.SMEM),
        scratch_shapes=[
            pltpu.VMEM((L, _F, _TL), jnp.float32),
            pltpu.VMEM((L, _F, _TL), jnp.float32),
            pltpu.VMEM((_F, _TL), jnp.float32),
            pltpu.VMEM((_F, _TL), jnp.float32),
            pltpu.VMEM((_F, _TL), jnp.float32),
            pltpu.VMEM((_F, _TL), jnp.float32),
            pltpu.SemaphoreType.DMA,
        ],
    )
    return pl.pallas_call(
        body,
        grid_spec=grid_spec,
        out_shape=jax.ShapeDtypeStruct((1,), jnp.float32),
    )


def kernel(u, i, t, last_basket, V_IL, V_LI, V_UL, V_LU, V_UI, V_IU):
    L = last_basket.shape[0]
    out = _fpmc_tc(L)(
        last_basket.astype(jnp.int32),
        jnp.asarray(i, jnp.int32)[None],
        jnp.asarray(u, jnp.int32)[None],
        jnp.asarray(t, jnp.int32)[None],
        V_IL.T, V_LI.T, V_UL.T, V_LU.T, V_UI.T, V_IU.T)
    return out[0]
